# baseline (speedup 1.0000x reference)
import functools

import jax
import jax.numpy as jnp
from jax import lax
from jax.experimental import pallas as pl
from jax.experimental.pallas import tpu as pltpu
from jax.experimental.pallas import tpu_sc as plsc

D = 32
L = 16
BB = 128
NT = 20
NBLK = 4


@functools.lru_cache(maxsize=None)
def _make_kernel(B):
    NC, NS = 2, 16
    NW = NC * NS
    CB = BB * NT
    assert B == NW * NBLK * CB

    mesh = plsc.VectorSubcoreMesh(core_axis_name="c", subcore_axis_name="s")

    @functools.partial(
        pl.kernel,
        mesh=mesh,
        out_type=jax.ShapeDtypeStruct((NT, D // 8, 16384 // BB, 8, BB), jnp.float32),
        compiler_params=pltpu.CompilerParams(
            use_tc_tiling_on_sc=False, needs_layout_passes=False
        ),
        scratch_types=[
            pltpu.VMEM((CB // BB, BB), jnp.int32),
            pltpu.VMEM((NT, BB), jnp.int32),
            pltpu.VMEM((NT, BB), jnp.int32),
            pltpu.VMEM((2, BB, D), jnp.float32),
            pltpu.VMEM((2, D // 8, 8, BB), jnp.float32),
            pltpu.VMEM((D,), jnp.float32),
            pltpu.SemaphoreType.DMA,
            pltpu.SemaphoreType.DMA,
            pltpu.SemaphoreType.DMA,
            pltpu.SemaphoreType.DMA,
        ],
    )
    def emb(idx_hbm, train_hbm, freeze_hbm, out_hbm,
            idx_v, adj_v, raw_v, rows_v, tiles_v, fz_v,
            gsem0, gsem1, wsem0, wsem1):
        gsem = [gsem0, gsem1]
        wsem = [wsem0, wsem1]
        wid = lax.axis_index("s") * NC + lax.axis_index("c")
        pltpu.sync_copy(freeze_hbm.at[0], fz_v)

        def fire(t, buf):
            pltpu.async_copy(
                train_hbm.at[adj_v.at[t]], rows_v.at[buf], gsem[buf]
            )

        def drain_gather(buf):
            pltpu.make_async_copy(
                train_hbm.at[pl.ds(0, BB)], rows_v.at[buf], gsem[buf]
            ).wait()

        def write_tiles(t, blk, buf):
            for cb in range(D // 8):
                pltpu.async_copy(
                    tiles_v.at[buf].at[cb],
                    out_hbm.at[t].at[cb].at[blk],
                    wsem[buf],
                )

        def drain_writes(buf):
            for cb in range(D // 8):
                pltpu.make_async_copy(
                    tiles_v.at[buf].at[cb],
                    out_hbm.at[0].at[cb].at[0],
                    wsem[buf],
                ).wait()

        def proc(t, blk, buf, anyz, guard_first):
            drain_gather(buf)

            @pl.when(anyz)
            def _():
                fz = [fz_v[pl.ds(k * L, L)] for k in range(D // L)]

                def fixg(k, carry):
                    v = raw_v[t, pl.ds(k * L, L)]
                    m = v == 0

                    @pl.when(jnp.any(m))
                    def _():
                        bvec = k * L + lax.iota(jnp.int32, L)
                        for col in range(D):
                            colv = jnp.full((L,), col, jnp.int32)
                            val = jnp.full((L,), fz[col // L][col % L],
                                           jnp.float32)
                            plsc.store_scatter(rows_v.at[buf], [bvec, colv],
                                               val, mask=m)

                    return carry

                lax.fori_loop(0, BB // L, fixg, jnp.int32(0))

            if guard_first:
                @pl.when(t >= 2)
                def _():
                    drain_writes(buf)
            else:
                drain_writes(buf)

            def transp(cc, carry):
                ccol = jnp.zeros((L,), jnp.int32) + cc
                for k in range(BB // L):
                    bvec = k * L + lax.iota(jnp.int32, L)
                    vals = plsc.load_gather(rows_v.at[buf], [bvec, ccol])
                    tiles_v[buf, cc // 8, cc % 8, pl.ds(k * L, L)] = vals
                return carry

            lax.fori_loop(0, D, transp, jnp.int32(0))
            write_tiles(t, blk, buf)

        for bk in range(NBLK):
            blk = wid * NBLK + bk
            pltpu.sync_copy(idx_hbm.at[pl.ds(blk * (CB // BB), CB // BB)], idx_v)

            def adjust(g, za):
                r = g // (BB // L)
                col = (g % (BB // L)) * L
                v = idx_v[r, pl.ds(col, L)]
                za = jnp.logical_or(za, v == 0)
                p = g * L + lax.iota(jnp.int32, L)
                tvec = p % NT
                bvec = p // NT
                plsc.store_scatter(raw_v, [tvec, bvec], v)
                plsc.store_scatter(adj_v, [tvec, bvec], jnp.maximum(v - 1, 0))
                return za

            za = lax.fori_loop(0, CB // L, adjust, jnp.zeros((L,), jnp.bool_))
            anyz = jnp.any(za)

            fire(0, 0)

            def pair(j, carry):
                t0 = 2 * j
                fire(t0 + 1, 1)
                proc(t0, blk, 0, anyz, guard_first=(bk == 0))

                @pl.when(t0 + 2 < NT)
                def _():
                    fire(t0 + 2, 0)

                proc(t0 + 1, blk, 1, anyz, guard_first=(bk == 0))
                return carry

            lax.fori_loop(0, NT // 2, pair, jnp.int32(0))

        drain_writes(0)
        drain_writes(1)

    return emb


def kernel(x, weight_train, weight_freeze):
    B = x.size
    xf = x.reshape(B // BB, BB).astype(jnp.int32)
    out5 = _make_kernel(B)(xf, weight_train, weight_freeze)
    return out5.transpose(2, 4, 0, 1, 3).reshape(16384, NT, D)


# speedup vs baseline: 1.3299x; 1.3299x over previous
import functools

import jax
import jax.numpy as jnp
from jax import lax
from jax.experimental import pallas as pl
from jax.experimental.pallas import tpu as pltpu
from jax.experimental.pallas import tpu_sc as plsc

D = 32
L = 16
BB = 128
NT = 20
NBLK = 4


@functools.lru_cache(maxsize=None)
def _make_kernel(B):
    NC, NS = 2, 16
    NW = NC * NS
    CB = BB * NT
    assert B == NW * NBLK * CB

    mesh = plsc.VectorSubcoreMesh(core_axis_name="c", subcore_axis_name="s")

    @functools.partial(
        pl.kernel,
        mesh=mesh,
        out_type=jax.ShapeDtypeStruct((NT, D // 8, 16384 // BB, 8, BB), jnp.float32),
        compiler_params=pltpu.CompilerParams(
            use_tc_tiling_on_sc=False, needs_layout_passes=False
        ),
        scratch_types=[
            pltpu.VMEM((CB // BB, BB), jnp.int32),
            pltpu.VMEM((NT, BB), jnp.int32),
            pltpu.VMEM((NT, BB), jnp.int32),
            pltpu.VMEM((4, BB, D), jnp.float32),
            pltpu.VMEM((4, D, 131), jnp.float32),
            pltpu.VMEM((D,), jnp.float32),
            pltpu.SemaphoreType.DMA,
            pltpu.SemaphoreType.DMA,
            pltpu.SemaphoreType.DMA,
            pltpu.SemaphoreType.DMA,
            pltpu.SemaphoreType.DMA,
            pltpu.SemaphoreType.DMA,
            pltpu.SemaphoreType.DMA,
            pltpu.SemaphoreType.DMA,
        ],
    )
    def emb(idx_hbm, train_hbm, freeze_hbm, out_hbm,
            idx_v, adj_v, raw_v, rows_v, tiles_v, fz_v,
            gsem0, gsem1, gsem2, gsem3, wsem0, wsem1, wsem2, wsem3):
        gsem = [gsem0, gsem1, gsem2, gsem3]
        wsem = [wsem0, wsem1, wsem2, wsem3]
        wid = lax.axis_index("s") * NC + lax.axis_index("c")
        pltpu.sync_copy(freeze_hbm.at[0], fz_v)

        def fire(t, buf):
            pltpu.async_copy(
                train_hbm.at[adj_v.at[t]], rows_v.at[buf], gsem[buf]
            )

        def drain_gather(buf):
            pltpu.make_async_copy(
                train_hbm.at[pl.ds(0, BB)], rows_v.at[buf], gsem[buf]
            ).wait()

        def write_tiles(t, blk, buf):
            for cb in range(D // 8):
                pltpu.async_copy(
                    tiles_v.at[buf].at[pl.ds(8 * cb, 8), pl.ds(0, BB)],
                    out_hbm.at[t].at[cb].at[blk],
                    wsem[buf],
                )

        def drain_writes(buf):
            for cb in range(D // 8):
                pltpu.make_async_copy(
                    tiles_v.at[buf].at[pl.ds(8 * cb, 8), pl.ds(0, BB)],
                    out_hbm.at[0].at[cb].at[0],
                    wsem[buf],
                ).wait()

        def proc(t, blk, buf, anyz, guard_first):
            drain_gather(buf)

            @pl.when(anyz)
            def _():
                fz = [fz_v[pl.ds(k * L, L)] for k in range(D // L)]

                def fixg(k, carry):
                    v = raw_v[t, pl.ds(k * L, L)]
                    m = v == 0

                    @pl.when(jnp.any(m))
                    def _():
                        bvec = k * L + lax.iota(jnp.int32, L)
                        for col in range(D):
                            colv = jnp.full((L,), col, jnp.int32)
                            val = jnp.full((L,), fz[col // L][col % L],
                                           jnp.float32)
                            plsc.store_scatter(rows_v.at[buf], [bvec, colv],
                                               val, mask=m)

                    return carry

                lax.fori_loop(0, BB // L, fixg, jnp.int32(0))

            if guard_first:
                @pl.when(t >= 4)
                def _():
                    drain_writes(buf)
            else:
                drain_writes(buf)

            c_lo = lax.iota(jnp.int32, L)
            c_hi = c_lo + L

            def transp(b, carry):
                bs = jnp.zeros((L,), jnp.int32) + b
                v0 = rows_v[buf, b, pl.ds(0, L)]
                v1 = rows_v[buf, b, pl.ds(L, L)]
                plsc.store_scatter(tiles_v.at[buf], [c_lo, bs], v0)
                plsc.store_scatter(tiles_v.at[buf], [c_hi, bs], v1)
                return carry

            lax.fori_loop(0, BB, transp, jnp.int32(0))
            write_tiles(t, blk, buf)

        for bk in range(NBLK):
            blk = wid * NBLK + bk
            pltpu.sync_copy(idx_hbm.at[pl.ds(blk * (CB // BB), CB // BB)], idx_v)

            def adjust(g, za):
                r = g // (BB // L)
                col = (g % (BB // L)) * L
                v = idx_v[r, pl.ds(col, L)]
                za = jnp.logical_or(za, v == 0)
                p = g * L + lax.iota(jnp.int32, L)
                tvec = p % NT
                bvec = p // NT
                plsc.store_scatter(raw_v, [tvec, bvec], v)
                plsc.store_scatter(adj_v, [tvec, bvec], jnp.maximum(v - 1, 0))
                return za

            za = lax.fori_loop(0, CB // L, adjust, jnp.zeros((L,), jnp.bool_))
            anyz = jnp.any(za)

            for s in range(4):
                fire(s, s)

            def quad(j, carry):
                for s in range(4):
                    t = 4 * j + s
                    proc(t, blk, s, anyz, guard_first=(bk == 0))

                    @pl.when(t + 4 < NT)
                    def _():
                        fire(t + 4, s)

                return carry

            lax.fori_loop(0, NT // 4, quad, jnp.int32(0))

        for s in range(4):
            drain_writes(s)

    return emb


def kernel(x, weight_train, weight_freeze):
    B = x.size
    xf = x.reshape(B // BB, BB).astype(jnp.int32)
    out5 = _make_kernel(B)(xf, weight_train, weight_freeze)
    return out5.transpose(2, 4, 0, 1, 3).reshape(16384, NT, D)


# speedup vs baseline: 1.9916x; 1.4976x over previous
import functools

import jax
import jax.numpy as jnp
from jax import lax
from jax.experimental import pallas as pl
from jax.experimental.pallas import tpu as pltpu
from jax.experimental.pallas import tpu_sc as plsc

D = 32
L = 16
BB = 128
NT = 20
NBLK = 4
NV = 999999
TS = (NV // BB) * BB
NC, NS = 2, 16
NW = NC * NS


@functools.lru_cache(maxsize=None)
def _make_detile():
    mesh = plsc.VectorSubcoreMesh(core_axis_name="c", subcore_axis_name="s")
    NCH = TS // BB
    PERW = NCH // NW
    EXTRA = NCH - PERW * NW

    @functools.partial(
        pl.kernel,
        mesh=mesh,
        out_type=jax.ShapeDtypeStruct((TS * D,), jnp.float32),
        compiler_params=pltpu.CompilerParams(
            use_tc_tiling_on_sc=True, needs_layout_passes=False
        ),
        scratch_types=[
            pltpu.VMEM((D, BB), jnp.float32),
            pltpu.VMEM((D, BB), jnp.float32),
            pltpu.VMEM((BB * D,), jnp.float32),
            pltpu.VMEM((BB * D,), jnp.float32),
            pltpu.SemaphoreType.DMA,
            pltpu.SemaphoreType.DMA,
            pltpu.SemaphoreType.DMA,
            pltpu.SemaphoreType.DMA,
        ],
    )
    def detile(wtT_hbm, out_hbm, tiles0, tiles1, flat0, flat1,
               isem0, isem1, osem0, osem1):
        tiles = [tiles0, tiles1]
        flat = [flat0, flat1]
        isem = [isem0, isem1]
        osem = [osem0, osem1]
        wid = lax.axis_index("s") * NC + lax.axis_index("c")
        base = wid * PERW
        iot = lax.iota(jnp.int32, L)

        def fire_in(j, buf):
            for cb in range(D // 8):
                pltpu.async_copy(
                    wtT_hbm.at[pl.ds(8 * cb, 8), pl.ds(BB * j, BB)],
                    tiles[buf].at[pl.ds(8 * cb, 8)],
                    isem[buf],
                )

        def drain_in(buf):
            for cb in range(D // 8):
                pltpu.make_async_copy(
                    wtT_hbm.at[pl.ds(0, 8), pl.ds(0, BB)],
                    tiles[buf].at[pl.ds(8 * cb, 8)],
                    isem[buf],
                ).wait()

        def write_out(j, buf):
            pltpu.async_copy(
                flat[buf], out_hbm.at[pl.ds(j * BB * D, BB * D)], osem[buf]
            )

        def drain_out(buf):
            pltpu.make_async_copy(
                flat[buf], out_hbm.at[pl.ds(0, BB * D)], osem[buf]
            ).wait()

        def transpose(buf):
            def body(c0, carry):
                cvec = (c0 + iot) & (D - 1)
                for k in range(BB // L):
                    rvec = k * L + iot
                    vals = plsc.load_gather(tiles[buf], [cvec, rvec])
                    plsc.store_scatter(flat[buf], [rvec * D + cvec], vals)
                return carry

            lax.fori_loop(0, D, body, jnp.int32(0))

        fire_in(base, 0)

        def pair(p, carry):
            j0 = base + 2 * p
            fire_in(j0 + 1, 1)
            drain_in(0)

            @pl.when(p > 0)
            def _():
                drain_out(0)

            transpose(0)
            write_out(j0, 0)

            @pl.when(2 * p + 2 < PERW)
            def _():
                fire_in(j0 + 2, 0)

            drain_in(1)

            @pl.when(p > 0)
            def _():
                drain_out(1)

            transpose(1)
            write_out(j0 + 1, 1)
            return carry

        lax.fori_loop(0, PERW // 2, pair, jnp.int32(0))

        @pl.when(wid < EXTRA)
        def _():
            j = NW * PERW + wid
            fire_in(j, 0)
            drain_in(0)
            drain_out(0)
            transpose(0)
            write_out(j, 0)

        drain_out(0)
        drain_out(1)

    return detile


@functools.lru_cache(maxsize=None)
def _make_gather(B):
    CB = BB * NT
    assert B == NW * NBLK * CB
    NTAIL = NV - TS

    mesh = plsc.VectorSubcoreMesh(core_axis_name="c", subcore_axis_name="s")

    @functools.partial(
        pl.kernel,
        mesh=mesh,
        out_type=jax.ShapeDtypeStruct((NT, D // 8, 16384 // BB, 8, BB), jnp.float32),
        compiler_params=pltpu.CompilerParams(
            use_tc_tiling_on_sc=False, needs_layout_passes=False
        ),
        scratch_types=[
            pltpu.VMEM((CB // BB, BB), jnp.int32),
            pltpu.VMEM((NT, BB), jnp.int32),
            pltpu.VMEM((NT, BB), jnp.int32),
            pltpu.VMEM((4, BB, D), jnp.float32),
            pltpu.VMEM((4, D, 131), jnp.float32),
            pltpu.VMEM((D,), jnp.float32),
            pltpu.VMEM((NV - TS, D), jnp.float32),
            pltpu.SemaphoreType.DMA,
            pltpu.SemaphoreType.DMA,
            pltpu.SemaphoreType.DMA,
            pltpu.SemaphoreType.DMA,
            pltpu.SemaphoreType.DMA,
            pltpu.SemaphoreType.DMA,
            pltpu.SemaphoreType.DMA,
            pltpu.SemaphoreType.DMA,
        ],
    )
    def emb(idx_hbm, tbl_hbm, tail_hbm, freeze_hbm, out_hbm,
            idx_v, adj_v, raw_v, rows_v, tiles_v, fz_v, tail_v,
            gsem0, gsem1, gsem2, gsem3, wsem0, wsem1, wsem2, wsem3):
        gsem = [gsem0, gsem1, gsem2, gsem3]
        wsem = [wsem0, wsem1, wsem2, wsem3]
        wid = lax.axis_index("s") * NC + lax.axis_index("c")
        pltpu.sync_copy(freeze_hbm.at[0], fz_v)
        pltpu.sync_copy(tail_hbm, tail_v)

        def fire(t, buf):
            pltpu.async_copy(
                tbl_hbm.at[adj_v.at[t]], rows_v.at[buf], gsem[buf]
            )

        def drain_gather(buf):
            pltpu.make_async_copy(
                tbl_hbm.at[pl.ds(0, BB)], rows_v.at[buf], gsem[buf]
            ).wait()

        def write_tiles(t, blk, buf):
            for cb in range(D // 8):
                pltpu.async_copy(
                    tiles_v.at[buf].at[pl.ds(8 * cb, 8), pl.ds(0, BB)],
                    out_hbm.at[t].at[cb].at[blk],
                    wsem[buf],
                )

        def drain_writes(buf):
            for cb in range(D // 8):
                pltpu.make_async_copy(
                    tiles_v.at[buf].at[pl.ds(8 * cb, 8), pl.ds(0, BB)],
                    out_hbm.at[0].at[cb].at[0],
                    wsem[buf],
                ).wait()

        def proc(t, blk, buf, anyfix, guard_first):
            drain_gather(buf)

            @pl.when(anyfix)
            def _():
                fz = [fz_v[pl.ds(k * L, L)] for k in range(D // L)]

                def fixg(k, carry):
                    v = raw_v[t, pl.ds(k * L, L)]
                    mz = v == 0
                    mt = v > TS
                    bvec = k * L + lax.iota(jnp.int32, L)

                    @pl.when(jnp.any(mz))
                    def _():
                        for col in range(D):
                            colv = jnp.full((L,), col, jnp.int32)
                            val = jnp.full((L,), fz[col // L][col % L],
                                           jnp.float32)
                            plsc.store_scatter(rows_v.at[buf], [bvec, colv],
                                               val, mask=mz)

                    @pl.when(jnp.any(mt))
                    def _():
                        rv = jnp.clip(v - 1 - TS, 0, NV - TS - 1)
                        for col in range(D):
                            colv = jnp.full((L,), col, jnp.int32)
                            vals = plsc.load_gather(tail_v, [rv, colv],
                                                    mask=mt)
                            plsc.store_scatter(rows_v.at[buf], [bvec, colv],
                                               vals, mask=mt)

                    return carry

                lax.fori_loop(0, BB // L, fixg, jnp.int32(0))

            if guard_first:
                @pl.when(t >= 4)
                def _():
                    drain_writes(buf)
            else:
                drain_writes(buf)

            c_lo = lax.iota(jnp.int32, L)
            c_hi = c_lo + L

            def transp(b, carry):
                bs = jnp.zeros((L,), jnp.int32) + b
                v0 = rows_v[buf, b, pl.ds(0, L)]
                v1 = rows_v[buf, b, pl.ds(L, L)]
                plsc.store_scatter(tiles_v.at[buf], [c_lo, bs], v0)
                plsc.store_scatter(tiles_v.at[buf], [c_hi, bs], v1)
                return carry

            lax.fori_loop(0, BB, transp, jnp.int32(0))
            write_tiles(t, blk, buf)

        for bk in range(NBLK):
            blk = wid * NBLK + bk
            pltpu.sync_copy(idx_hbm.at[pl.ds(blk * (CB // BB), CB // BB)], idx_v)

            def adjust(g, za):
                r = g // (BB // L)
                col = (g % (BB // L)) * L
                v = idx_v[r, pl.ds(col, L)]
                za = jnp.logical_or(za, jnp.logical_or(v == 0, v > TS))
                p = g * L + lax.iota(jnp.int32, L)
                tvec = p % NT
                bvec = p // NT
                adj = jnp.maximum(v - 1, 0)
                adj = jnp.where(v > TS, 0, adj)
                plsc.store_scatter(raw_v, [tvec, bvec], v)
                plsc.store_scatter(adj_v, [tvec, bvec], adj)
                return za

            za = lax.fori_loop(0, CB // L, adjust, jnp.zeros((L,), jnp.bool_))
            anyfix = jnp.any(za)

            for s in range(4):
                fire(s, s)

            def quad(j, carry):
                for s in range(4):
                    t = 4 * j + s
                    proc(t, blk, s, anyfix, guard_first=(bk == 0))

                    @pl.when(t + 4 < NT)
                    def _():
                        fire(t + 4, s)

                return carry

            lax.fori_loop(0, NT // 4, quad, jnp.int32(0))

        for s in range(4):
            drain_writes(s)

    return emb


def kernel(x, weight_train, weight_freeze):
    B = x.size
    xf = x.reshape(B // BB, BB).astype(jnp.int32)
    tbl = _make_detile()(weight_train.T).reshape(TS, D)
    tail = weight_train[TS:]
    out5 = _make_gather(B)(xf, tbl, tail, weight_freeze)
    return out5.transpose(2, 4, 0, 1, 3).reshape(16384, NT, D)


# speedup vs baseline: 1.9923x; 1.0003x over previous
import functools

import jax
import jax.numpy as jnp
from jax import lax
from jax.experimental import pallas as pl
from jax.experimental.pallas import tpu as pltpu
from jax.experimental.pallas import tpu_sc as plsc

D = 32
L = 16
BB = 128
NT = 20
NBLK = 4
NV = 999999
TS = (NV // BB) * BB
NC, NS = 2, 16
NW = NC * NS


@functools.lru_cache(maxsize=None)
def _make_detile():
    mesh = plsc.VectorSubcoreMesh(core_axis_name="c", subcore_axis_name="s")
    NCH = TS // BB
    PERW = NCH // NW
    EXTRA = NCH - PERW * NW

    @functools.partial(
        pl.kernel,
        mesh=mesh,
        out_type=jax.ShapeDtypeStruct((TS * D,), jnp.float32),
        compiler_params=pltpu.CompilerParams(
            use_tc_tiling_on_sc=True, needs_layout_passes=False
        ),
        scratch_types=[
            pltpu.VMEM((D, BB), jnp.float32),
            pltpu.VMEM((D, BB), jnp.float32),
            pltpu.VMEM((BB * D,), jnp.float32),
            pltpu.VMEM((BB * D,), jnp.float32),
            pltpu.SemaphoreType.DMA,
            pltpu.SemaphoreType.DMA,
            pltpu.SemaphoreType.DMA,
            pltpu.SemaphoreType.DMA,
        ],
    )
    def detile(wtT_hbm, out_hbm, tiles0, tiles1, flat0, flat1,
               isem0, isem1, osem0, osem1):
        tiles = [tiles0, tiles1]
        flat = [flat0, flat1]
        isem = [isem0, isem1]
        osem = [osem0, osem1]
        wid = lax.axis_index("s") * NC + lax.axis_index("c")
        base = wid * PERW
        iot = lax.iota(jnp.int32, L)

        def fire_in(j, buf):
            for cb in range(D // 8):
                pltpu.async_copy(
                    wtT_hbm.at[pl.ds(8 * cb, 8), pl.ds(BB * j, BB)],
                    tiles[buf].at[pl.ds(8 * cb, 8)],
                    isem[buf],
                )

        def drain_in(buf):
            for cb in range(D // 8):
                pltpu.make_async_copy(
                    wtT_hbm.at[pl.ds(0, 8), pl.ds(0, BB)],
                    tiles[buf].at[pl.ds(8 * cb, 8)],
                    isem[buf],
                ).wait()

        def write_out(j, buf):
            pltpu.async_copy(
                flat[buf], out_hbm.at[pl.ds(j * BB * D, BB * D)], osem[buf]
            )

        def drain_out(buf):
            pltpu.make_async_copy(
                flat[buf], out_hbm.at[pl.ds(0, BB * D)], osem[buf]
            ).wait()

        rvecs = [k * L + iot for k in range(BB // L)]
        rbases = [k * L * D + iot * D for k in range(BB // L)]

        def transpose(buf):
            def body(c0, carry):
                cvec = (c0 + iot) & (D - 1)
                for k in range(BB // L):
                    vals = plsc.load_gather(tiles[buf], [cvec, rvecs[k]])
                    plsc.store_scatter(flat[buf], [rbases[k] + cvec], vals)
                return carry

            lax.fori_loop(0, D, body, jnp.int32(0))

        fire_in(base, 0)

        def pair(p, carry):
            j0 = base + 2 * p
            fire_in(j0 + 1, 1)
            drain_in(0)

            @pl.when(p > 0)
            def _():
                drain_out(0)

            transpose(0)
            write_out(j0, 0)

            @pl.when(2 * p + 2 < PERW)
            def _():
                fire_in(j0 + 2, 0)

            drain_in(1)

            @pl.when(p > 0)
            def _():
                drain_out(1)

            transpose(1)
            write_out(j0 + 1, 1)
            return carry

        lax.fori_loop(0, PERW // 2, pair, jnp.int32(0))

        @pl.when(wid < EXTRA)
        def _():
            j = NW * PERW + wid
            fire_in(j, 0)
            drain_in(0)
            drain_out(0)
            transpose(0)
            write_out(j, 0)

        drain_out(0)
        drain_out(1)

    return detile


@functools.lru_cache(maxsize=None)
def _make_gather(B):
    CB = BB * NT
    assert B == NW * NBLK * CB
    NTAIL = NV - TS

    mesh = plsc.VectorSubcoreMesh(core_axis_name="c", subcore_axis_name="s")

    @functools.partial(
        pl.kernel,
        mesh=mesh,
        out_type=jax.ShapeDtypeStruct((NT, D // 8, 16384 // BB, 8, BB), jnp.float32),
        compiler_params=pltpu.CompilerParams(
            use_tc_tiling_on_sc=False, needs_layout_passes=False
        ),
        scratch_types=[
            pltpu.VMEM((CB // BB, BB), jnp.int32),
            pltpu.VMEM((NT, BB), jnp.int32),
            pltpu.VMEM((NT, BB), jnp.int32),
            pltpu.VMEM((4, BB, D), jnp.float32),
            pltpu.VMEM((4, D, 131), jnp.float32),
            pltpu.VMEM((D,), jnp.float32),
            pltpu.VMEM((NV - TS, D), jnp.float32),
            pltpu.SemaphoreType.DMA,
            pltpu.SemaphoreType.DMA,
            pltpu.SemaphoreType.DMA,
            pltpu.SemaphoreType.DMA,
            pltpu.SemaphoreType.DMA,
            pltpu.SemaphoreType.DMA,
            pltpu.SemaphoreType.DMA,
            pltpu.SemaphoreType.DMA,
        ],
    )
    def emb(idx_hbm, tbl_hbm, tail_hbm, freeze_hbm, out_hbm,
            idx_v, adj_v, raw_v, rows_v, tiles_v, fz_v, tail_v,
            gsem0, gsem1, gsem2, gsem3, wsem0, wsem1, wsem2, wsem3):
        gsem = [gsem0, gsem1, gsem2, gsem3]
        wsem = [wsem0, wsem1, wsem2, wsem3]
        wid = lax.axis_index("s") * NC + lax.axis_index("c")
        pltpu.sync_copy(freeze_hbm.at[0], fz_v)
        pltpu.sync_copy(tail_hbm, tail_v)

        def fire(t, buf):
            pltpu.async_copy(
                tbl_hbm.at[adj_v.at[t]], rows_v.at[buf], gsem[buf]
            )

        def drain_gather(buf):
            pltpu.make_async_copy(
                tbl_hbm.at[pl.ds(0, BB)], rows_v.at[buf], gsem[buf]
            ).wait()

        def write_tiles(t, blk, buf):
            for cb in range(D // 8):
                pltpu.async_copy(
                    tiles_v.at[buf].at[pl.ds(8 * cb, 8), pl.ds(0, BB)],
                    out_hbm.at[t].at[cb].at[blk],
                    wsem[buf],
                )

        def drain_writes(buf):
            for cb in range(D // 8):
                pltpu.make_async_copy(
                    tiles_v.at[buf].at[pl.ds(8 * cb, 8), pl.ds(0, BB)],
                    out_hbm.at[0].at[cb].at[0],
                    wsem[buf],
                ).wait()

        def proc(t, blk, buf, anyfix, guard_first):
            drain_gather(buf)

            @pl.when(anyfix)
            def _():
                fz = [fz_v[pl.ds(k * L, L)] for k in range(D // L)]

                def fixg(k, carry):
                    v = raw_v[t, pl.ds(k * L, L)]
                    mz = v == 0
                    mt = v > TS
                    bvec = k * L + lax.iota(jnp.int32, L)

                    @pl.when(jnp.any(mz))
                    def _():
                        for col in range(D):
                            colv = jnp.full((L,), col, jnp.int32)
                            val = jnp.full((L,), fz[col // L][col % L],
                                           jnp.float32)
                            plsc.store_scatter(rows_v.at[buf], [bvec, colv],
                                               val, mask=mz)

                    @pl.when(jnp.any(mt))
                    def _():
                        rv = jnp.clip(v - 1 - TS, 0, NV - TS - 1)
                        for col in range(D):
                            colv = jnp.full((L,), col, jnp.int32)
                            vals = plsc.load_gather(tail_v, [rv, colv],
                                                    mask=mt)
                            plsc.store_scatter(rows_v.at[buf], [bvec, colv],
                                               vals, mask=mt)

                    return carry

                lax.fori_loop(0, BB // L, fixg, jnp.int32(0))

            if guard_first:
                @pl.when(t >= 4)
                def _():
                    drain_writes(buf)
            else:
                drain_writes(buf)

            c_lo = lax.iota(jnp.int32, L)
            c_hi = c_lo + L

            def transp(b, carry):
                bs = jnp.zeros((L,), jnp.int32) + b
                v0 = rows_v[buf, b, pl.ds(0, L)]
                v1 = rows_v[buf, b, pl.ds(L, L)]
                plsc.store_scatter(tiles_v.at[buf], [c_lo, bs], v0)
                plsc.store_scatter(tiles_v.at[buf], [c_hi, bs], v1)
                return carry

            lax.fori_loop(0, BB, transp, jnp.int32(0))
            write_tiles(t, blk, buf)

        for bk in range(NBLK):
            blk = wid * NBLK + bk
            pltpu.sync_copy(idx_hbm.at[pl.ds(blk * (CB // BB), CB // BB)], idx_v)

            def adjust(g, za):
                r = g // (BB // L)
                col = (g % (BB // L)) * L
                v = idx_v[r, pl.ds(col, L)]
                za = jnp.logical_or(za, jnp.logical_or(v == 0, v > TS))
                p = g * L + lax.iota(jnp.int32, L)
                tvec = p % NT
                bvec = p // NT
                adj = jnp.maximum(v - 1, 0)
                adj = jnp.where(v > TS, 0, adj)
                plsc.store_scatter(raw_v, [tvec, bvec], v)
                plsc.store_scatter(adj_v, [tvec, bvec], adj)
                return za

            za = lax.fori_loop(0, CB // L, adjust, jnp.zeros((L,), jnp.bool_))
            anyfix = jnp.any(za)

            for s in range(4):
                fire(s, s)

            def quad(j, carry):
                for s in range(4):
                    t = 4 * j + s
                    proc(t, blk, s, anyfix, guard_first=(bk == 0))

                    @pl.when(t + 4 < NT)
                    def _():
                        fire(t + 4, s)

                return carry

            lax.fori_loop(0, NT // 4, quad, jnp.int32(0))

        for s in range(4):
            drain_writes(s)

    return emb


def kernel(x, weight_train, weight_freeze):
    B = x.size
    xf = x.reshape(B // BB, BB).astype(jnp.int32)
    tbl = _make_detile()(weight_train.T).reshape(TS, D)
    tail = weight_train[TS:]
    out5 = _make_gather(B)(xf, tbl, tail, weight_freeze)
    return out5.transpose(2, 4, 0, 1, 3).reshape(16384, NT, D)
